# baseline (speedup 1.0000x reference)
import jax
import jax.numpy as jnp
from jax.experimental import pallas as pl
from jax.experimental.pallas import tpu as pltpu

_N = 207
_IN = 2
_U = 128
_B = 64
_BC = 16
_NCHUNK = _B // _BC
_SEQ = 12
_C = 5
_NM = 3
_NB = _N * _BC


def _dcrnn_body(xt_ref, idx_ref, s_ref,
                w0ri_ref, w0rh_ref, b0r_ref, w0ui_ref, w0uh_ref, b0u_ref,
                w0ci_ref, w0ch_ref, b0c_ref,
                w1ri_ref, w1rh_ref, b1r_ref, w1ui_ref, w1uh_ref, b1u_ref,
                w1ci_ref, w1ch_ref, b1c_ref,
                fcw_ref, fcb_ref,
                out_ref,
                h0_ref, h1_ref, s2c_ref):
    f32 = jnp.float32
    bc = pl.program_id(0)
    t = pl.program_id(1)

    @pl.when(jnp.logical_and(bc == 0, t == 0))
    def _init_s2c():
        S0 = s_ref[:]
        eye = (jax.lax.broadcasted_iota(jnp.int32, (_N, _N), 0)
               == jax.lax.broadcasted_iota(jnp.int32, (_N, _N), 1)).astype(f32)
        s2c_ref[:] = 2.0 * jnp.dot(S0, S0, preferred_element_type=f32) - eye

    @pl.when(t == 0)
    def _init():
        h0_ref[:] = jnp.zeros((_N, _BC, _U), f32)
        h1_ref[:] = jnp.zeros((_N, _BC, _U), f32)
        out_ref[:] = jnp.zeros((_BC, _C), f32)

    S = s_ref[:]
    S2c = s2c_ref[:]

    bf16 = jnp.bfloat16
    S_b = S.astype(bf16)
    S2c_b = S2c.astype(bf16)

    def gconv(xin_r, h_r, wi_ref, wh_ref, b_ref):
        def p(k):
            return (jnp.dot(xin_r, wi_ref[k], preferred_element_type=f32)
                    + jnp.dot(h_r, wh_ref[k], preferred_element_type=f32))
        acc = p(0).reshape(_N, _BC * _U)
        acc = acc + jnp.dot(S_b, p(1).reshape(_N, _BC * _U).astype(bf16),
                            preferred_element_type=f32)
        acc = acc + jnp.dot(S2c_b, p(2).reshape(_N, _BC * _U).astype(bf16),
                            preferred_element_type=f32)
        return acc.reshape(_N, _BC, _U) + b_ref[:]

    def cell(xin_r, h_ref, wri, wrh, br, wui, wuh, bu, wci, wch, bcb):
        h3 = h_ref[:]
        h_r = h3.reshape(_NB, _U).astype(bf16)
        r = jax.nn.sigmoid(gconv(xin_r, h_r, wri, wrh, br))
        u = jax.nn.sigmoid(gconv(xin_r, h_r, wui, wuh, bu))
        rh_r = (r * h3).reshape(_NB, _U).astype(bf16)
        c = jnp.tanh(gconv(xin_r, rh_r, wci, wch, bcb))
        hn = u * h3 + (1.0 - u) * c
        h_ref[:] = hn
        return hn

    xin_r = xt_ref[0].reshape(_NB, _IN).astype(bf16)
    h0n = cell(xin_r, h0_ref, w0ri_ref, w0rh_ref, b0r_ref,
               w0ui_ref, w0uh_ref, b0u_ref, w0ci_ref, w0ch_ref, b0c_ref)
    h1n = cell(h0n.reshape(_NB, _U).astype(bf16), h1_ref,
               w1ri_ref, w1rh_ref, b1r_ref,
               w1ui_ref, w1uh_ref, b1u_ref, w1ci_ref, w1ch_ref, b1c_ref)

    lastv = jnp.maximum(h1n, 0.0).reshape(_NB, _U)
    logits = jnp.dot(lastv, fcw_ref[:], preferred_element_type=f32)
    pool = jnp.max(logits.reshape(_N, _BC, _C), axis=0) + fcb_ref[:]
    mask = idx_ref[:] == t
    out_ref[:] = jnp.where(mask, pool, out_ref[:])


def _split_w(W, fin):
    return W.reshape(fin, _NM, -1).transpose(1, 0, 2)


def kernel(input_seq, seq_lengths, supports, Wg0, bg0, Wc0, bc0,
           Wg1, bg1, Wc1, bc1, fc_w, fc_b):
    f32 = jnp.float32
    xt = jnp.transpose(input_seq, (1, 2, 0, 3)).astype(f32)
    idx = jnp.clip(seq_lengths.astype(jnp.int32) - 1, 0, _SEQ - 1)
    idx = jnp.broadcast_to(idx.reshape(_B, 1), (_B, _C)).astype(jnp.int32)

    wg0 = _split_w(Wg0, _IN + _U)
    wc0 = _split_w(Wc0, _IN + _U)
    wg1 = _split_w(Wg1, _U + _U)
    wc1 = _split_w(Wc1, _U + _U)

    bf16 = jnp.bfloat16

    def parts(wg, wc, bg, bc, fin_x):
        return (
            wg[:, :fin_x, :_U].astype(bf16), wg[:, fin_x:, :_U].astype(bf16),
            bg[:_U].reshape(1, 1, _U),
            wg[:, :fin_x, _U:].astype(bf16), wg[:, fin_x:, _U:].astype(bf16),
            bg[_U:].reshape(1, 1, _U),
            wc[:, :fin_x, :].astype(bf16), wc[:, fin_x:, :].astype(bf16),
            bc.reshape(1, 1, _U),
        )

    cell0 = parts(wg0, wc0, bg0, bc0, _IN)
    cell1 = parts(wg1, wc1, bg1, bc1, _U)

    args = (
        xt, idx, supports.astype(f32),
        *cell0, *cell1,
        fc_w.astype(f32), fc_b.reshape(1, _C),
    )

    def const_spec(a):
        nd = a.ndim
        return pl.BlockSpec(a.shape, lambda bc, t, _nd=nd: (0,) * _nd)

    in_specs = [pl.BlockSpec((1, _N, _BC, _IN), lambda bc, t: (t, 0, bc, 0)),
                pl.BlockSpec((_BC, _C), lambda bc, t: (bc, 0))]
    in_specs += [const_spec(a) for a in args[2:]]

    out = pl.pallas_call(
        _dcrnn_body,
        grid=(_NCHUNK, _SEQ),
        in_specs=in_specs,
        out_specs=pl.BlockSpec((_BC, _C), lambda bc, t: (bc, 0)),
        scratch_shapes=[
            pltpu.VMEM((_N, _BC, _U), f32),
            pltpu.VMEM((_N, _BC, _U), f32),
            pltpu.VMEM((_N, _N), f32),
        ],
        out_shape=jax.ShapeDtypeStruct((_B, _C), f32),
        compiler_params=pltpu.CompilerParams(
            dimension_semantics=("arbitrary", "arbitrary"),
            vmem_limit_bytes=63 * 1024 * 1024,
        ),
    )(*args)
    return out


# speedup vs baseline: 1.3306x; 1.3306x over previous
import jax
import jax.numpy as jnp
from jax.experimental import pallas as pl
from jax.experimental.pallas import tpu as pltpu

_N = 207
_IN = 2
_U = 128
_B = 64
_BC = 16
_NCHUNK = _B // _BC
_SEQ = 12
_C = 5
_NM = 3
_NB = _N * _BC
_RU = 2 * _U
_WRU = _NM * _RU
_WC = _NM * _U
_WALL = _WRU + _WC


def _dcrnn_body(xt_ref, idx_ref, s_ref,
                w0in_ref, w0hru_ref, b0ru_ref, w0hc_ref, b0c_ref,
                w1in_ref, w1hru_ref, b1ru_ref, w1hc_ref, b1c_ref,
                fcw_ref, fcb_ref,
                out_ref,
                h0_ref, h1_ref, s2c_ref):
    f32 = jnp.float32
    bc = pl.program_id(0)
    t = pl.program_id(1)

    @pl.when(jnp.logical_and(bc == 0, t == 0))
    def _init_s2c():
        S0 = s_ref[:]
        eye = (jax.lax.broadcasted_iota(jnp.int32, (_N, _N), 0)
               == jax.lax.broadcasted_iota(jnp.int32, (_N, _N), 1)).astype(f32)
        s2c_ref[:] = 2.0 * jnp.dot(S0, S0, preferred_element_type=f32) - eye

    @pl.when(t == 0)
    def _init():
        h0_ref[:] = jnp.zeros((_N, _BC, _U), f32)
        h1_ref[:] = jnp.zeros((_N, _BC, _U), f32)
        out_ref[:] = jnp.zeros((_BC, _C), f32)

    S = s_ref[:]
    S2c = s2c_ref[:]

    def chebyshev(p, w):
        def nf(m):
            return m.reshape(_N, _BC * w)
        acc = nf(p[:, :w])
        acc = acc + jnp.dot(S, nf(p[:, w:2 * w]), preferred_element_type=f32)
        acc = acc + jnp.dot(S2c, nf(p[:, 2 * w:]), preferred_element_type=f32)
        return acc

    def cell(px, h_ref, whru_ref, bru_ref, whc_ref, bc_ref):
        h3 = h_ref[:]
        h_r = h3.reshape(_NB, _U)
        p_ru = px[:, :_WRU] + jnp.dot(h_r, whru_ref[:],
                                      preferred_element_type=f32)
        acc = chebyshev(p_ru, _RU).reshape(_N, _BC, _RU) + bru_ref[:]
        g = jax.nn.sigmoid(acc)
        r = g[:, :, :_U]
        u = g[:, :, _U:]
        rh_r = (r * h3).reshape(_NB, _U)
        p_c = px[:, _WRU:] + jnp.dot(rh_r, whc_ref[:],
                                     preferred_element_type=f32)
        accc = chebyshev(p_c, _U).reshape(_N, _BC, _U) + bc_ref[:]
        c = jnp.tanh(accc)
        hn = u * h3 + (1.0 - u) * c
        h_ref[:] = hn
        return hn

    xin_r = xt_ref[0].reshape(_NB, _IN)
    px0 = jnp.dot(xin_r, w0in_ref[:], preferred_element_type=f32)
    h0n = cell(px0, h0_ref, w0hru_ref, b0ru_ref, w0hc_ref, b0c_ref)
    px1 = jnp.dot(h0n.reshape(_NB, _U), w1in_ref[:],
                  preferred_element_type=f32)
    h1n = cell(px1, h1_ref, w1hru_ref, b1ru_ref, w1hc_ref, b1c_ref)

    lastv = jnp.maximum(h1n, 0.0).reshape(_NB, _U)
    logits = jnp.dot(lastv, fcw_ref[:], preferred_element_type=f32)
    pool = jnp.max(logits.reshape(_N, _BC, _C), axis=0) + fcb_ref[:]
    mask = idx_ref[:] == t
    out_ref[:] = jnp.where(mask, pool, out_ref[:])


def _split_w(W, fin):
    return W.reshape(fin, _NM, -1).transpose(1, 0, 2)


def kernel(input_seq, seq_lengths, supports, Wg0, bg0, Wc0, bc0,
           Wg1, bg1, Wc1, bc1, fc_w, fc_b):
    f32 = jnp.float32
    xt = jnp.transpose(input_seq, (1, 2, 0, 3)).astype(f32)
    idx = jnp.clip(seq_lengths.astype(jnp.int32) - 1, 0, _SEQ - 1)
    idx = jnp.broadcast_to(idx.reshape(_B, 1), (_B, _C)).astype(jnp.int32)

    def pack(Wg, Wc, bg, bcv, fin_x):
        wg = _split_w(Wg, fin_x + _U)
        wc = _split_w(Wc, fin_x + _U)
        w_in = jnp.concatenate(
            [wg[0, :fin_x], wg[1, :fin_x], wg[2, :fin_x],
             wc[0, :fin_x], wc[1, :fin_x], wc[2, :fin_x]], axis=1)
        w_hru = jnp.concatenate(
            [wg[0, fin_x:], wg[1, fin_x:], wg[2, fin_x:]], axis=1)
        w_hc = jnp.concatenate(
            [wc[0, fin_x:], wc[1, fin_x:], wc[2, fin_x:]], axis=1)
        return (w_in, w_hru, bg.reshape(1, 1, _RU),
                w_hc, bcv.reshape(1, 1, _U))

    cell0 = pack(Wg0, Wc0, bg0, bc0, _IN)
    cell1 = pack(Wg1, Wc1, bg1, bc1, _U)

    args = (
        xt, idx, supports.astype(f32),
        *cell0, *cell1,
        fc_w.astype(f32), fc_b.reshape(1, _C),
    )

    def const_spec(a):
        nd = a.ndim
        return pl.BlockSpec(a.shape, lambda bc, t, _nd=nd: (0,) * _nd)

    in_specs = [pl.BlockSpec((1, _N, _BC, _IN), lambda bc, t: (t, 0, bc, 0)),
                pl.BlockSpec((_BC, _C), lambda bc, t: (bc, 0))]
    in_specs += [const_spec(a) for a in args[2:]]

    out = pl.pallas_call(
        _dcrnn_body,
        grid=(_NCHUNK, _SEQ),
        in_specs=in_specs,
        out_specs=pl.BlockSpec((_BC, _C), lambda bc, t: (bc, 0)),
        scratch_shapes=[
            pltpu.VMEM((_N, _BC, _U), f32),
            pltpu.VMEM((_N, _BC, _U), f32),
            pltpu.VMEM((_N, _N), f32),
        ],
        out_shape=jax.ShapeDtypeStruct((_B, _C), f32),
        compiler_params=pltpu.CompilerParams(
            dimension_semantics=("arbitrary", "arbitrary"),
            vmem_limit_bytes=63 * 1024 * 1024,
        ),
    )(*args)
    return out


# speedup vs baseline: 1.3330x; 1.0018x over previous
import jax
import jax.numpy as jnp
from jax.experimental import pallas as pl
from jax.experimental.pallas import tpu as pltpu

_N = 207
_IN = 2
_U = 128
_B = 64
_BC = 16
_NCHUNK = _B // _BC
_SEQ = 12
_C = 5
_NM = 3
_NB = _N * _BC
_RU = 2 * _U
_WRU = _NM * _RU
_WC = _NM * _U
_WALL = _WRU + _WC


def _dcrnn_body(xt_ref, idx_ref, s_ref,
                w0in_ref, w0hru_ref, b0ru_ref, w0hc_ref, b0c_ref,
                w1in_ref, w1hru_ref, b1ru_ref, w1hc_ref, b1c_ref,
                fcw_ref, fcb_ref,
                out_ref,
                h0_ref, h1_ref, s2c_ref):
    f32 = jnp.float32
    bc = pl.program_id(0)
    t = pl.program_id(1)

    @pl.when(t == 0)
    def _init_s2c():
        S0 = s_ref[:]
        eye = (jax.lax.broadcasted_iota(jnp.int32, (_N, _N), 0)
               == jax.lax.broadcasted_iota(jnp.int32, (_N, _N), 1)).astype(f32)
        s2c_ref[:] = 2.0 * jnp.dot(S0, S0, preferred_element_type=f32) - eye

    @pl.when(t == 0)
    def _init():
        h0_ref[:] = jnp.zeros((_N, _BC, _U), f32)
        h1_ref[:] = jnp.zeros((_N, _BC, _U), f32)
        out_ref[:] = jnp.zeros((_BC, _C), f32)

    S = s_ref[:]
    S2c = s2c_ref[:]

    def chebyshev(p, w):
        def nf(m):
            return m.reshape(_N, _BC * w)
        acc = nf(p[:, :w])
        acc = acc + jnp.dot(S, nf(p[:, w:2 * w]), preferred_element_type=f32)
        acc = acc + jnp.dot(S2c, nf(p[:, 2 * w:]), preferred_element_type=f32)
        return acc

    def cell(px, h_ref, whru_ref, bru_ref, whc_ref, bc_ref):
        h3 = h_ref[:]
        h_r = h3.reshape(_NB, _U)
        p_ru = px[:, :_WRU] + jnp.dot(h_r, whru_ref[:],
                                      preferred_element_type=f32)
        acc = chebyshev(p_ru, _RU).reshape(_N, _BC, _RU) + bru_ref[:]
        g = jax.nn.sigmoid(acc)
        r = g[:, :, :_U]
        u = g[:, :, _U:]
        rh_r = (r * h3).reshape(_NB, _U)
        p_c = px[:, _WRU:] + jnp.dot(rh_r, whc_ref[:],
                                     preferred_element_type=f32)
        accc = chebyshev(p_c, _U).reshape(_N, _BC, _U) + bc_ref[:]
        c = jnp.tanh(accc)
        hn = u * h3 + (1.0 - u) * c
        h_ref[:] = hn
        return hn

    xin_r = xt_ref[0].reshape(_NB, _IN)
    px0 = jnp.dot(xin_r, w0in_ref[:], preferred_element_type=f32)
    h0n = cell(px0, h0_ref, w0hru_ref, b0ru_ref, w0hc_ref, b0c_ref)
    px1 = jnp.dot(h0n.reshape(_NB, _U), w1in_ref[:],
                  preferred_element_type=f32)
    h1n = cell(px1, h1_ref, w1hru_ref, b1ru_ref, w1hc_ref, b1c_ref)

    lastv = jnp.maximum(h1n, 0.0).reshape(_NB, _U)
    logits = jnp.dot(lastv, fcw_ref[:], preferred_element_type=f32)
    pool = jnp.max(logits.reshape(_N, _BC, _C), axis=0) + fcb_ref[:]
    mask = idx_ref[:] == t
    out_ref[:] = jnp.where(mask, pool, out_ref[:])


def _split_w(W, fin):
    return W.reshape(fin, _NM, -1).transpose(1, 0, 2)


def kernel(input_seq, seq_lengths, supports, Wg0, bg0, Wc0, bc0,
           Wg1, bg1, Wc1, bc1, fc_w, fc_b):
    f32 = jnp.float32
    xt = jnp.transpose(input_seq, (1, 2, 0, 3)).astype(f32)
    idx = jnp.clip(seq_lengths.astype(jnp.int32) - 1, 0, _SEQ - 1)
    idx = jnp.broadcast_to(idx.reshape(_B, 1), (_B, _C)).astype(jnp.int32)

    def pack(Wg, Wc, bg, bcv, fin_x):
        wg = _split_w(Wg, fin_x + _U)
        wc = _split_w(Wc, fin_x + _U)
        w_in = jnp.concatenate(
            [wg[0, :fin_x], wg[1, :fin_x], wg[2, :fin_x],
             wc[0, :fin_x], wc[1, :fin_x], wc[2, :fin_x]], axis=1)
        w_hru = jnp.concatenate(
            [wg[0, fin_x:], wg[1, fin_x:], wg[2, fin_x:]], axis=1)
        w_hc = jnp.concatenate(
            [wc[0, fin_x:], wc[1, fin_x:], wc[2, fin_x:]], axis=1)
        return (w_in, w_hru, bg.reshape(1, 1, _RU),
                w_hc, bcv.reshape(1, 1, _U))

    cell0 = pack(Wg0, Wc0, bg0, bc0, _IN)
    cell1 = pack(Wg1, Wc1, bg1, bc1, _U)

    args = (
        xt, idx, supports.astype(f32),
        *cell0, *cell1,
        fc_w.astype(f32), fc_b.reshape(1, _C),
    )

    def const_spec(a):
        nd = a.ndim
        return pl.BlockSpec(a.shape, lambda bc, t, _nd=nd: (0,) * _nd)

    in_specs = [pl.BlockSpec((1, _N, _BC, _IN), lambda bc, t: (t, 0, bc, 0)),
                pl.BlockSpec((_BC, _C), lambda bc, t: (bc, 0))]
    in_specs += [const_spec(a) for a in args[2:]]

    out = pl.pallas_call(
        _dcrnn_body,
        grid=(_NCHUNK, _SEQ),
        in_specs=in_specs,
        out_specs=pl.BlockSpec((_BC, _C), lambda bc, t: (bc, 0)),
        scratch_shapes=[
            pltpu.VMEM((_N, _BC, _U), f32),
            pltpu.VMEM((_N, _BC, _U), f32),
            pltpu.VMEM((_N, _N), f32),
        ],
        out_shape=jax.ShapeDtypeStruct((_B, _C), f32),
        compiler_params=pltpu.CompilerParams(
            dimension_semantics=("parallel", "arbitrary"),
            vmem_limit_bytes=63 * 1024 * 1024,
        ),
    )(*args)
    return out


# speedup vs baseline: 3.2439x; 2.4336x over previous
import jax
import jax.numpy as jnp
from jax.experimental import pallas as pl
from jax.experimental.pallas import tpu as pltpu

_N = 207
_IN = 2
_U = 128
_B = 64
_BC = 16
_NCHUNK = _B // _BC
_SEQ = 12
_C = 5
_NM = 3
_NB = _N * _BC
_RU = 2 * _U
_WRU = _NM * _RU
_WC = _NM * _U
_WALL = _WRU + _WC


def _dcrnn_body(xt_ref, idx_ref, s_ref,
                w0in_ref, w0hru_ref, b0ru_ref, w0hc_ref, b0c_ref,
                w1in_ref, w1hru_ref, b1ru_ref, w1hc_ref, b1c_ref,
                fcw_ref, fcb_ref,
                out_ref,
                h0_ref, h1_ref, s2c_ref):
    f32 = jnp.float32
    bc = pl.program_id(0)
    t = pl.program_id(1)

    @pl.when(t == 0)
    def _init_s2c():
        S0 = s_ref[:]
        eye = (jax.lax.broadcasted_iota(jnp.int32, (_N, _N), 0)
               == jax.lax.broadcasted_iota(jnp.int32, (_N, _N), 1)).astype(f32)
        s2c_ref[:] = 2.0 * jnp.dot(S0, S0, preferred_element_type=f32) - eye

    @pl.when(t == 0)
    def _init():
        h0_ref[:] = jnp.zeros((_N, _BC, _U), f32)
        h1_ref[:] = jnp.zeros((_N, _BC, _U), f32)
        out_ref[:] = jnp.zeros((_BC, _C), f32)

    S = s_ref[:]
    S2c = s2c_ref[:]

    def chebyshev(p, w):
        def nf(m):
            return m.reshape(_N, _BC * w)
        acc = jnp.dot(S, nf(p[:, w:2 * w]), preferred_element_type=f32)
        acc = acc + jnp.dot(S2c, nf(p[:, 2 * w:]), preferred_element_type=f32)
        return p[:, :w].reshape(_N, _BC, w) + acc.reshape(_N, _BC, w)

    def cell(px, h_ref, whru_ref, bru_ref, whc_ref, bc_ref):
        h3 = h_ref[:]
        h_r = h3.reshape(_NB, _U)
        p_ru = px[:, :_WRU] + jnp.dot(h_r, whru_ref[:],
                                      preferred_element_type=f32)
        acc = chebyshev(p_ru, _RU) + bru_ref[:]
        g = jax.nn.sigmoid(acc)
        r = g[:, :, :_U]
        u = g[:, :, _U:]
        rh_r = (r * h3).reshape(_NB, _U)
        p_c = px[:, _WRU:] + jnp.dot(rh_r, whc_ref[:],
                                     preferred_element_type=f32)
        accc = chebyshev(p_c, _U) + bc_ref[:]
        c = jnp.tanh(accc)
        hn = u * h3 + (1.0 - u) * c
        h_ref[:] = hn
        return hn

    xin_r = xt_ref[0].reshape(_NB, _IN)
    px0 = jnp.dot(xin_r, w0in_ref[:], preferred_element_type=f32)
    h0n = cell(px0, h0_ref, w0hru_ref, b0ru_ref, w0hc_ref, b0c_ref)
    px1 = jnp.dot(h0n.reshape(_NB, _U), w1in_ref[:],
                  preferred_element_type=f32)
    h1n = cell(px1, h1_ref, w1hru_ref, b1ru_ref, w1hc_ref, b1c_ref)

    lastv = jnp.maximum(h1n, 0.0).reshape(_NB, _U)
    logits = jnp.dot(lastv, fcw_ref[:], preferred_element_type=f32)
    pool = jnp.max(logits.reshape(_N, _BC, _C), axis=0) + fcb_ref[:]
    mask = idx_ref[:] == t
    out_ref[:] = jnp.where(mask, pool, out_ref[:])


def _split_w(W, fin):
    return W.reshape(fin, _NM, -1).transpose(1, 0, 2)


def kernel(input_seq, seq_lengths, supports, Wg0, bg0, Wc0, bc0,
           Wg1, bg1, Wc1, bc1, fc_w, fc_b):
    f32 = jnp.float32
    xt = jnp.transpose(input_seq, (1, 2, 0, 3)).astype(f32)
    idx = jnp.clip(seq_lengths.astype(jnp.int32) - 1, 0, _SEQ - 1)
    idx = jnp.broadcast_to(idx.reshape(_B, 1), (_B, _C)).astype(jnp.int32)

    def pack(Wg, Wc, bg, bcv, fin_x):
        wg = _split_w(Wg, fin_x + _U)
        wc = _split_w(Wc, fin_x + _U)
        w_in = jnp.concatenate(
            [wg[0, :fin_x], wg[1, :fin_x], wg[2, :fin_x],
             wc[0, :fin_x], wc[1, :fin_x], wc[2, :fin_x]], axis=1)
        w_hru = jnp.concatenate(
            [wg[0, fin_x:], wg[1, fin_x:], wg[2, fin_x:]], axis=1)
        w_hc = jnp.concatenate(
            [wc[0, fin_x:], wc[1, fin_x:], wc[2, fin_x:]], axis=1)
        return (w_in, w_hru, bg.reshape(1, 1, _RU),
                w_hc, bcv.reshape(1, 1, _U))

    cell0 = pack(Wg0, Wc0, bg0, bc0, _IN)
    cell1 = pack(Wg1, Wc1, bg1, bc1, _U)

    args = (
        xt, idx, supports.astype(f32),
        *cell0, *cell1,
        fc_w.astype(f32), fc_b.reshape(1, _C),
    )

    def const_spec(a):
        nd = a.ndim
        return pl.BlockSpec(a.shape, lambda bc, t, _nd=nd: (0,) * _nd)

    in_specs = [pl.BlockSpec((1, _N, _BC, _IN), lambda bc, t: (t, 0, bc, 0)),
                pl.BlockSpec((_BC, _C), lambda bc, t: (bc, 0))]
    in_specs += [const_spec(a) for a in args[2:]]

    out = pl.pallas_call(
        _dcrnn_body,
        grid=(_NCHUNK, _SEQ),
        in_specs=in_specs,
        out_specs=pl.BlockSpec((_BC, _C), lambda bc, t: (bc, 0)),
        scratch_shapes=[
            pltpu.VMEM((_N, _BC, _U), f32),
            pltpu.VMEM((_N, _BC, _U), f32),
            pltpu.VMEM((_N, _N), f32),
        ],
        out_shape=jax.ShapeDtypeStruct((_B, _C), f32),
        compiler_params=pltpu.CompilerParams(
            dimension_semantics=("parallel", "arbitrary"),
            vmem_limit_bytes=63 * 1024 * 1024,
        ),
    )(*args)
    return out
